# baseline (speedup 1.0000x reference)
import functools

import jax
import jax.numpy as jnp
from jax import lax
from jax.experimental import pallas as pl
from jax.experimental.pallas import tpu as pltpu
from jax.experimental.pallas import tpu_sc as plsc

_R_LO, _R_HI = 1.0, 5.0
_NC = 2
_NS = 16
_NW = _NC * _NS
_G = 8
_NB = 32768


def _tc_pack_groups(table_t):
    E, N = table_t.shape
    eye = jnp.eye(E, dtype=jnp.float32)

    def body(t_ref, i_ref, o_ref):
        o_ref[:] = lax.dot_general(
            t_ref[:], i_ref[:], (((0,), (0,)), ((), ())),
            preferred_element_type=jnp.float32)

    packed = pl.pallas_call(
        body,
        grid=((N + _NB - 1) // _NB,),
        in_specs=[
            pl.BlockSpec((E, _NB), lambda c: (0, c)),
            pl.BlockSpec((E, E), lambda c: (0, 0)),
        ],
        out_specs=pl.BlockSpec((_NB, E), lambda c: (c, 0)),
        out_shape=jax.ShapeDtypeStruct((N, E), jnp.float32),
    )(table_t, eye)
    return packed.reshape(N // _G, _G * E)


def _sc_gather_groups(idx, tab_g):
    B = idx.shape[0]
    b_per_w = B // _NW
    h = b_per_w // 2
    mesh = plsc.VectorSubcoreMesh(core_axis_name="c", subcore_axis_name="s")

    @functools.partial(
        pl.kernel,
        mesh=mesh,
        out_type=jax.ShapeDtypeStruct((B, 128), jnp.float32),
        scratch_types=[
            pltpu.VMEM((b_per_w,), jnp.int32),
            pltpu.VMEM((b_per_w,), jnp.int32),
            pltpu.VMEM((h, 128), jnp.float32),
            pltpu.VMEM((h, 128), jnp.float32),
            pltpu.SemaphoreType.DMA,
            pltpu.SemaphoreType.DMA,
        ],
    )
    def gather_kernel(idx_hbm, tab_hbm, out_hbm, idx_v, gid_v, buf0, buf1,
                      sem0, sem1):
        wid = lax.axis_index("s") * _NC + lax.axis_index("c")
        base = wid * b_per_w
        pltpu.sync_copy(idx_hbm.at[pl.ds(base, b_per_w)], idx_v)
        for j in range(b_per_w // 16):
            sl = pl.ds(j * 16, 16)
            gid_v[sl] = lax.shift_right_logical(idx_v[sl], 3)
        c0 = pltpu.async_copy(tab_hbm.at[gid_v.at[pl.ds(0, h)]], buf0, sem0)
        c1 = pltpu.async_copy(tab_hbm.at[gid_v.at[pl.ds(h, h)]], buf1, sem1)
        c0.wait()
        pltpu.sync_copy(buf0, out_hbm.at[pl.ds(base, h)])
        c1.wait()
        pltpu.sync_copy(buf1, out_hbm.at[pl.ds(base + h, h)])

    return gather_kernel(idx, tab_g)


def _tc_mlp(X, ug, ig, w1u_t, w1i_t, wg_t, bg2, w1g_t, b12, w2t, b22):
    B = X.shape[0]

    def body(x_ref, ug_ref, ig_ref, w1u_ref, w1i_ref, wg_ref, bg_ref,
             w1g_ref, b1_ref, w2_ref, b2_ref, o_ref):
        Bb = x_ref.shape[0]
        lane = lax.broadcasted_iota(jnp.int32, (Bb, 128), 1)
        ksel = lax.shift_right_logical(lane, 4)
        mu = jnp.bitwise_and(x_ref[:, 0:1], _G - 1)
        mi = jnp.bitwise_and(x_ref[:, 1:2], _G - 1)
        um = ug_ref[:] * (ksel == mu).astype(jnp.float32)
        im = ig_ref[:] * (ksel == mi).astype(jnp.float32)
        g = x_ref[:, 2:].astype(jnp.float32)
        eg = jnp.dot(g, wg_ref[:], preferred_element_type=jnp.float32) + bg_ref[:]
        hh = (jnp.dot(um, w1u_ref[:], preferred_element_type=jnp.float32)
              + jnp.dot(im, w1i_ref[:], preferred_element_type=jnp.float32)
              + jnp.dot(eg, w1g_ref[:], preferred_element_type=jnp.float32)
              + b1_ref[:])
        hh = jnp.maximum(hh, 0.0)
        p = jnp.dot(hh, w2_ref[:], preferred_element_type=jnp.float32) + b2_ref[:]
        o_ref[:] = jax.nn.sigmoid(p) * (_R_HI - _R_LO) + _R_LO

    Bb = 2048
    full = lambda shape: pl.BlockSpec(shape, lambda i: (0, 0))
    return pl.pallas_call(
        body,
        grid=(B // Bb,),
        in_specs=[
            pl.BlockSpec((Bb, X.shape[1]), lambda i: (i, 0)),
            pl.BlockSpec((Bb, 128), lambda i: (i, 0)),
            pl.BlockSpec((Bb, 128), lambda i: (i, 0)),
            full(w1u_t.shape),
            full(w1i_t.shape),
            full(wg_t.shape),
            full(bg2.shape),
            full(w1g_t.shape),
            full(b12.shape),
            full(w2t.shape),
            full(b22.shape),
        ],
        out_specs=pl.BlockSpec((Bb, 1), lambda i: (i, 0)),
        out_shape=jax.ShapeDtypeStruct((B, 1), jnp.float32),
    )(X, ug, ig, w1u_t, w1i_t, wg_t, bg2, w1g_t, b12, w2t, b22)


def kernel(X, user_table, item_table, Wg, bg, W1, b1, W2, b2):
    xu = X[:, 0]
    xi = X[:, 1]
    utab_g = _tc_pack_groups(user_table.T)
    ug = _sc_gather_groups(xu, utab_g)
    itab_g = _tc_pack_groups(item_table.T)
    ig = _sc_gather_groups(xi, itab_g)

    w1t = W1.T
    w1u_t = jnp.tile(w1t[0:16], (_G, 1))
    w1i_t = jnp.tile(w1t[16:32], (_G, 1))
    wg_t = Wg.T
    bg2 = bg.reshape(1, -1)
    w1g_t = w1t[32:48]
    b12 = b1.reshape(1, -1)
    w2t = W2.T
    b22 = b2.reshape(1, -1)
    return _tc_mlp(X, ug, ig, w1u_t, w1i_t, wg_t, bg2, w1g_t, b12, w2t, b22)


# speedup vs baseline: 1.8211x; 1.8211x over previous
import functools

import jax
import jax.numpy as jnp
from jax import lax
from jax.experimental import pallas as pl
from jax.experimental.pallas import tpu as pltpu
from jax.experimental.pallas import tpu_sc as plsc

_R_LO, _R_HI = 1.0, 5.0
_NC = 2
_NS = 16
_NW = _NC * _NS
_G = 8
_NB = 32768


def _tc_pack_groups(table_t):
    E, N = table_t.shape

    def body(t_ref, o_ref):
        y = t_ref[:].T.reshape(-1, _G, E)
        for q in range(_G):
            o_ref[:, q * E:(q + 1) * E] = y[:, q, :]

    return pl.pallas_call(
        body,
        grid=((N + _NB - 1) // _NB,),
        in_specs=[pl.BlockSpec((E, _NB), lambda c: (0, c))],
        out_specs=pl.BlockSpec((_NB // _G, _G * E), lambda c: (c, 0)),
        out_shape=jax.ShapeDtypeStruct((N // _G, _G * E), jnp.float32),
    )(table_t)


def _sc_gather_groups(idx, tab_g):
    B = idx.shape[0]
    b_per_w = B // _NW
    h = b_per_w // 2
    mesh = plsc.VectorSubcoreMesh(core_axis_name="c", subcore_axis_name="s")

    @functools.partial(
        pl.kernel,
        mesh=mesh,
        out_type=jax.ShapeDtypeStruct((B, 128), jnp.float32),
        scratch_types=[
            pltpu.VMEM((b_per_w,), jnp.int32),
            pltpu.VMEM((b_per_w,), jnp.int32),
            pltpu.VMEM((h, 128), jnp.float32),
            pltpu.VMEM((h, 128), jnp.float32),
            pltpu.SemaphoreType.DMA,
            pltpu.SemaphoreType.DMA,
        ],
    )
    def gather_kernel(idx_hbm, tab_hbm, out_hbm, idx_v, gid_v, buf0, buf1,
                      sem0, sem1):
        wid = lax.axis_index("s") * _NC + lax.axis_index("c")
        base = wid * b_per_w
        pltpu.sync_copy(idx_hbm.at[pl.ds(base, b_per_w)], idx_v)
        for j in range(b_per_w // 16):
            sl = pl.ds(j * 16, 16)
            gid_v[sl] = lax.shift_right_logical(idx_v[sl], 3)
        c0 = pltpu.async_copy(tab_hbm.at[gid_v.at[pl.ds(0, h)]], buf0, sem0)
        c1 = pltpu.async_copy(tab_hbm.at[gid_v.at[pl.ds(h, h)]], buf1, sem1)
        c0.wait()
        pltpu.sync_copy(buf0, out_hbm.at[pl.ds(base, h)])
        c1.wait()
        pltpu.sync_copy(buf1, out_hbm.at[pl.ds(base + h, h)])

    return gather_kernel(idx, tab_g)


def _tc_mlp(X, ug, ig, w1u_t, w1i_t, wg_t, bg2, w1g_t, b12, w2t, b22):
    B = X.shape[0]

    def body(x_ref, ug_ref, ig_ref, w1u_ref, w1i_ref, wg_ref, bg_ref,
             w1g_ref, b1_ref, w2_ref, b2_ref, o_ref):
        Bb = x_ref.shape[0]
        lane = lax.broadcasted_iota(jnp.int32, (Bb, 128), 1)
        ksel = lax.shift_right_logical(lane, 4)
        mu = jnp.bitwise_and(x_ref[:, 0:1], _G - 1)
        mi = jnp.bitwise_and(x_ref[:, 1:2], _G - 1)
        um = ug_ref[:] * (ksel == mu).astype(jnp.float32)
        im = ig_ref[:] * (ksel == mi).astype(jnp.float32)
        g = x_ref[:, 2:].astype(jnp.float32)
        eg = jnp.dot(g, wg_ref[:], preferred_element_type=jnp.float32) + bg_ref[:]
        hh = (jnp.dot(um, w1u_ref[:], preferred_element_type=jnp.float32)
              + jnp.dot(im, w1i_ref[:], preferred_element_type=jnp.float32)
              + jnp.dot(eg, w1g_ref[:], preferred_element_type=jnp.float32)
              + b1_ref[:])
        hh = jnp.maximum(hh, 0.0)
        p = jnp.dot(hh, w2_ref[:], preferred_element_type=jnp.float32) + b2_ref[:]
        o_ref[:] = jax.nn.sigmoid(p) * (_R_HI - _R_LO) + _R_LO

    Bb = 2048
    full = lambda shape: pl.BlockSpec(shape, lambda i: (0, 0))
    return pl.pallas_call(
        body,
        grid=(B // Bb,),
        in_specs=[
            pl.BlockSpec((Bb, X.shape[1]), lambda i: (i, 0)),
            pl.BlockSpec((Bb, 128), lambda i: (i, 0)),
            pl.BlockSpec((Bb, 128), lambda i: (i, 0)),
            full(w1u_t.shape),
            full(w1i_t.shape),
            full(wg_t.shape),
            full(bg2.shape),
            full(w1g_t.shape),
            full(b12.shape),
            full(w2t.shape),
            full(b22.shape),
        ],
        out_specs=pl.BlockSpec((Bb, 1), lambda i: (i, 0)),
        out_shape=jax.ShapeDtypeStruct((B, 1), jnp.float32),
    )(X, ug, ig, w1u_t, w1i_t, wg_t, bg2, w1g_t, b12, w2t, b22)


def kernel(X, user_table, item_table, Wg, bg, W1, b1, W2, b2):
    xu = X[:, 0]
    xi = X[:, 1]
    utab_g = _tc_pack_groups(user_table.T)
    ug = _sc_gather_groups(xu, utab_g)
    itab_g = _tc_pack_groups(item_table.T)
    ig = _sc_gather_groups(xi, itab_g)

    w1t = W1.T
    w1u_t = jnp.tile(w1t[0:16], (_G, 1))
    w1i_t = jnp.tile(w1t[16:32], (_G, 1))
    wg_t = Wg.T
    bg2 = bg.reshape(1, -1)
    w1g_t = w1t[32:48]
    b12 = b1.reshape(1, -1)
    w2t = W2.T
    b22 = b2.reshape(1, -1)
    return _tc_mlp(X, ug, ig, w1u_t, w1i_t, wg_t, bg2, w1g_t, b12, w2t, b22)
